# baseline (speedup 1.0000x reference)
import functools

import jax
import jax.numpy as jnp
from jax import lax
from jax.experimental import pallas as pl
from jax.experimental.pallas import tpu as pltpu
from jax.experimental.pallas import tpu_sc as plsc

N = 10000
E = 160000
D = 128
DE = 4
DS = 16
NW = 64
H = 4

NHP = 40960
NWORK = 32
BE = 2000

CE1 = 5000
NC1 = E // CE1
CPW1 = NC1 // NWORK
CE2 = 128
NC2 = E // CE2


def _mesh():
    return plsc.VectorSubcoreMesh(core_axis_name="c", subcore_axis_name="s")


def _lanes():
    return lax.iota(jnp.int32, 16)


def _zero_1d(ref, n):
    z = jnp.zeros((16,), jnp.float32)

    def body(i, _):
        ref[pl.ds(i * 16, 16)] = z
        return 0

    lax.fori_loop(0, n // 16, body, 0)


def _rsqrt_quake(x):
    i = plsc.bitcast(x, jnp.int32)
    i = 0x5F3759DF - lax.shift_right_arithmetic(i, 1)
    y = plsc.bitcast(i, jnp.float32)
    for _ in range(3):
        y = y * (1.5 - 0.5 * x * y * y)
    return y


RBS = 8192
RSUB = RBS // 16


def _block_reduce(acc, shared, tmp, red, out_at_c, s):
    for b in range(NHP // RBS):
        pltpu.sync_copy(acc.at[pl.ds(b * RBS, RBS)], shared.at[s])
        plsc.subcore_barrier()
        off = s * RSUB
        _zero_1d(red, RSUB)
        for j in range(16):
            pltpu.sync_copy(shared.at[j, pl.ds(off, RSUB)], tmp)

            def radd(i, _):
                red[pl.ds(i * 16, 16)] = (red[pl.ds(i * 16, 16)]
                                          + tmp[pl.ds(i * 16, 16)])
                return 0

            lax.fori_loop(0, RSUB // 16, radd, 0)
        pltpu.sync_copy(red, out_at_c.at[pl.ds(b * RBS + off, RSUB)])
        plsc.subcore_barrier()


def _kg_body(srci_hbm, dsti_hbm, nf_hbm, srcf_out, dstf_out,
             idx0, idx1, buf0, buf1, sem0, sem1):
    c = lax.axis_index("c")
    s = lax.axis_index("s")
    wid = c * 16 + s
    t = lax.shift_right_arithmetic(wid, 4)
    g = lax.bitwise_and(wid, 15)

    def run(idx_hbm, out_hbm):
        def pair(ci2, _):
            cid_a = (2 * ci2) * 16 + g
            cid_b = (2 * ci2 + 1) * 16 + g
            pltpu.sync_copy(idx_hbm.at[pl.ds(cid_a * CE2, CE2)], idx0)
            a = pltpu.async_copy(nf_hbm.at[idx0], buf0, sem0)
            pltpu.sync_copy(idx_hbm.at[pl.ds(cid_b * CE2, CE2)], idx1)
            b = pltpu.async_copy(nf_hbm.at[idx1], buf1, sem1)
            a.wait()
            pltpu.sync_copy(buf0, out_hbm.at[pl.ds(cid_a * CE2, CE2), :])
            b.wait()
            pltpu.sync_copy(buf1, out_hbm.at[pl.ds(cid_b * CE2, CE2), :])
            return 0

        lax.fori_loop(0, (NC2 // 16) // 2, pair, 0)

        @pl.when(g < NC2 % 16)
        def _():
            cid = (NC2 // 16) * 16 + g
            pltpu.sync_copy(idx_hbm.at[pl.ds(cid * CE2, CE2)], idx0)
            pltpu.async_copy(nf_hbm.at[idx0], buf0, sem0).wait()
            pltpu.sync_copy(buf0, out_hbm.at[pl.ds(cid * CE2, CE2), :])

    @pl.when(t == 0)
    def _():
        run(srci_hbm, srcf_out)

    @pl.when(t == 1)
    def _():
        run(dsti_hbm, dstf_out)


def _kg_stage(edge_src, edge_dst, node_feat):
    f = functools.partial(
        pl.kernel,
        mesh=_mesh(),
        compiler_params=pltpu.CompilerParams(needs_layout_passes=False),
        out_type=[
            jax.ShapeDtypeStruct((E, D), jnp.float32),
            jax.ShapeDtypeStruct((E, D), jnp.float32),
        ],
        scratch_types=[
            pltpu.VMEM((CE2,), jnp.int32),
            pltpu.VMEM((CE2,), jnp.int32),
            pltpu.VMEM((CE2, D), jnp.float32),
            pltpu.VMEM((CE2, D), jnp.float32),
            pltpu.SemaphoreType.DMA,
            pltpu.SemaphoreType.DMA,
        ],
    )
    return f(_kg_body)(edge_src, edge_dst, node_feat)


def _ks_body(expq_hbm, dst_hbm, s_out, acc, dstv, qv, tmp, red, shared):
    c = lax.axis_index("c")
    s = lax.axis_index("s")
    wid = c * 16 + s
    lanes = _lanes()
    _zero_1d(acc, NHP)

    def chunk(ci, _):
        base = (wid * CPW1 + ci) * CE1
        pltpu.sync_copy(dst_hbm.at[pl.ds(base, CE1)], dstv)
        pltpu.sync_copy(expq_hbm.at[pl.ds(base * H, CE1 * H)], qv)

        def grp(g):
            p = g * 16 + lanes
            el = lax.shift_right_arithmetic(p, 2)
            h = lax.bitwise_and(p, 3)
            d = plsc.bitcast(plsc.load_gather(dstv, [el]), jnp.int32)
            idx = d * 4 + h
            val = qv[pl.ds(g * 16, 16)]
            plsc.addupdate_scatter(acc, [idx], val)

        plsc.parallel_loop(0, CE1 * H // 16, unroll=4)(grp)
        return 0

    lax.fori_loop(0, CPW1, chunk, 0)

    _block_reduce(acc, shared, tmp, red, s_out.at[c], s)


def _ks_stage(expq_flat, dst_flat):
    f = functools.partial(
        pl.kernel,
        mesh=_mesh(),
        compiler_params=pltpu.CompilerParams(needs_layout_passes=False),
        out_type=jax.ShapeDtypeStruct((2, NHP), jnp.float32),
        scratch_types=[
            pltpu.VMEM((NHP,), jnp.float32),
            pltpu.VMEM((CE1,), jnp.float32),
            pltpu.VMEM((CE1 * H,), jnp.float32),
            pltpu.VMEM((RSUB,), jnp.float32),
            pltpu.VMEM((RSUB,), jnp.float32),
            pltpu.VMEM_SHARED((16, RBS), jnp.float32),
        ],
    )
    return f(_ks_body)(expq_flat, dst_flat)


def _kz_body(expq_hbm, dst_hbm, cut_hbm, s_part, z_out, r_out,
             stab, zacc, dstv, qv, cutv, tmp, red, shared):
    c = lax.axis_index("c")
    s = lax.axis_index("s")
    wid = c * 16 + s
    lanes = _lanes()

    pltpu.sync_copy(s_part.at[0], stab)
    for b in range(NHP // RSUB):
        pltpu.sync_copy(s_part.at[1, pl.ds(b * RSUB, RSUB)], tmp)

        def badd(i, _):
            o = b * RSUB + i * 16
            stab[pl.ds(o, 16)] = stab[pl.ds(o, 16)] + tmp[pl.ds(i * 16, 16)]
            return 0

        lax.fori_loop(0, RSUB // 16, badd, 0)

    _zero_1d(zacc, NHP)

    def chunk(ci, _):
        base = (wid * CPW1 + ci) * CE1
        pltpu.sync_copy(dst_hbm.at[pl.ds(base, CE1)], dstv)
        pltpu.sync_copy(expq_hbm.at[pl.ds(base * H, CE1 * H)], qv)
        pltpu.sync_copy(cut_hbm.at[pl.ds(base, CE1)], cutv)

        def grp(g):
            p = g * 16 + lanes
            el = lax.shift_right_arithmetic(p, 2)
            h = lax.bitwise_and(p, 3)
            d = plsc.bitcast(plsc.load_gather(dstv, [el]), jnp.int32)
            idx = d * 4 + h
            sv = plsc.load_gather(stab, [idx])
            q = qv[pl.ds(g * 16, 16)]
            r = q / sv
            qv[pl.ds(g * 16, 16)] = r
            r2 = r * r
            cu = plsc.load_gather(cutv, [el])
            plsc.addupdate_scatter(zacc, [idx], cu * r2 * r2)

        plsc.parallel_loop(0, CE1 * H // 16, unroll=4)(grp)
        pltpu.sync_copy(qv, r_out.at[pl.ds(base * H, CE1 * H)])
        return 0

    lax.fori_loop(0, CPW1, chunk, 0)

    _block_reduce(zacc, shared, tmp, red, z_out.at[c], s)


def _kz_stage(expq_flat, dst_flat, cut, s_part):
    f = functools.partial(
        pl.kernel,
        mesh=_mesh(),
        compiler_params=pltpu.CompilerParams(needs_layout_passes=False),
        out_type=[
            jax.ShapeDtypeStruct((2, NHP), jnp.float32),
            jax.ShapeDtypeStruct((E * H,), jnp.float32),
        ],
        scratch_types=[
            pltpu.VMEM((NHP,), jnp.float32),
            pltpu.VMEM((NHP,), jnp.float32),
            pltpu.VMEM((CE1,), jnp.float32),
            pltpu.VMEM((CE1 * H,), jnp.float32),
            pltpu.VMEM((CE1,), jnp.float32),
            pltpu.VMEM((RSUB,), jnp.float32),
            pltpu.VMEM((RSUB,), jnp.float32),
            pltpu.VMEM_SHARED((16, RBS), jnp.float32),
        ],
    )
    return f(_kz_body)(expq_flat, dst_flat, cut, s_part)


def _ksc_body(r_hbm, cut_hbm, dst_hbm, z_part, scale_out,
              ztab, tmp, rv, cutv, dstv):
    c = lax.axis_index("c")
    s = lax.axis_index("s")
    wid = c * 16 + s
    lanes = _lanes()

    pltpu.sync_copy(z_part.at[0], ztab)
    ZB = 4096
    for b in range(NHP // ZB):
        pltpu.sync_copy(z_part.at[1, pl.ds(b * ZB, ZB)], tmp)

        def badd(i, _):
            o = b * ZB + i * 16
            x = ztab[pl.ds(o, 16)] + tmp[pl.ds(i * 16, 16)]
            xc = jnp.where(x == 0.0, 1.0, x)
            ztab[pl.ds(o, 16)] = _rsqrt_quake(xc)
            return 0

        lax.fori_loop(0, ZB // 16, badd, 0)

    def chunk(ci, _):
        base = (wid * CPW1 + ci) * CE1
        pltpu.sync_copy(r_hbm.at[pl.ds(base * H, CE1 * H)], rv)
        pltpu.sync_copy(cut_hbm.at[pl.ds(base, CE1)], cutv)
        pltpu.sync_copy(dst_hbm.at[pl.ds(base, CE1)], dstv)

        def grp(g):
            p = g * 16 + lanes
            el = lax.shift_right_arithmetic(p, 2)
            h = lax.bitwise_and(p, 3)
            d = plsc.bitcast(plsc.load_gather(dstv, [el]), jnp.int32)
            iz = plsc.load_gather(ztab, [d * 4 + h])
            cu = plsc.load_gather(cutv, [el])
            sq = cu * _rsqrt_quake(jnp.where(cu == 0.0, 1.0, cu))
            rr = rv[pl.ds(g * 16, 16)]
            rv[pl.ds(g * 16, 16)] = rr * rr * sq * iz

        plsc.parallel_loop(0, CE1 * H // 16, unroll=4)(grp)
        pltpu.sync_copy(rv, scale_out.at[pl.ds(base * H, CE1 * H)])
        return 0

    lax.fori_loop(0, CPW1, chunk, 0)


def _ksc_stage(r_flat, cut, dst_bits, z_part):
    f = functools.partial(
        pl.kernel,
        mesh=_mesh(),
        compiler_params=pltpu.CompilerParams(needs_layout_passes=False),
        out_type=jax.ShapeDtypeStruct((E * H,), jnp.float32),
        scratch_types=[
            pltpu.VMEM((NHP,), jnp.float32),
            pltpu.VMEM((4096,), jnp.float32),
            pltpu.VMEM((CE1 * H,), jnp.float32),
            pltpu.VMEM((CE1,), jnp.float32),
            pltpu.VMEM((CE1,), jnp.float32),
        ],
    )
    return f(_ksc_body)(r_flat, cut, dst_bits, z_part)


NHALF = 5000
NROWS = 5120


def _kw_body(evw_hbm, idx2_hbm, out_part, evv, idxv, evv1, idxv1,
             sem0, sem1, nodeacc):
    c = lax.axis_index("c")
    s = lax.axis_index("s")
    zv = jnp.zeros((16,), jnp.float32)

    def zb(t, _):
        evv[lax.shift_right_arithmetic(t, 3),
            pl.ds(lax.bitwise_and(t, 7) * 16, 16)] = zv
        return 0

    lax.fori_loop(0, 128 * 8, zb, 0)
    for k, nr in ((0, 128), (1, 128), (2, 64)):
        pltpu.sync_copy(evv.at[pl.ds(0, nr), :],
                        nodeacc.at[pl.ds(s * 320 + k * 128, nr), :])
    plsc.subcore_barrier()

    def pair(t, _):
        base_a = ((2 * t) * 16 + s) * CE2
        base_b = ((2 * t + 1) * 16 + s) * CE2
        a0 = pltpu.async_copy(evw_hbm.at[pl.ds(base_a, CE2), :], evv, sem0)
        a1 = pltpu.async_copy(idx2_hbm.at[c, pl.ds(base_a, CE2)], idxv, sem0)
        b0 = pltpu.async_copy(evw_hbm.at[pl.ds(base_b, CE2), :], evv1, sem1)
        b1 = pltpu.async_copy(idx2_hbm.at[c, pl.ds(base_b, CE2)], idxv1, sem1)
        a0.wait()
        a1.wait()
        pltpu.sync_copy(evv, nodeacc.at[idxv], add=True)
        b0.wait()
        b1.wait()
        pltpu.sync_copy(evv1, nodeacc.at[idxv1], add=True)
        return 0

    lax.fori_loop(0, NC2 // 32, pair, 0)

    @pl.when(s < NC2 % 16)
    def _():
        base = ((NC2 // 16) * 16 + s) * CE2
        pltpu.sync_copy(evw_hbm.at[pl.ds(base, CE2), :], evv)
        pltpu.sync_copy(idx2_hbm.at[c, pl.ds(base, CE2)], idxv)
        pltpu.sync_copy(evv, nodeacc.at[idxv], add=True)

    plsc.subcore_barrier()
    for k, nr in ((0, 128), (1, 128), (2, 64)):
        o = s * 320 + k * 128
        pltpu.sync_copy(nodeacc.at[pl.ds(o, nr), :],
                        out_part.at[c, pl.ds(o, nr), :])


def _kw_stage(evw, idx2):
    f = functools.partial(
        pl.kernel,
        mesh=_mesh(),
        compiler_params=pltpu.CompilerParams(needs_layout_passes=False),
        out_type=jax.ShapeDtypeStruct((2, NROWS, D), jnp.float32),
        scratch_types=[
            pltpu.VMEM((CE2, D), jnp.float32),
            pltpu.VMEM((CE2,), jnp.int32),
            pltpu.VMEM((CE2, D), jnp.float32),
            pltpu.VMEM((CE2,), jnp.int32),
            pltpu.SemaphoreType.DMA,
            pltpu.SemaphoreType.DMA,
            pltpu.VMEM_SHARED((NROWS, D), jnp.float32),
        ],
    )
    return f(_kw_body)(evw, idx2)


def _edge_kernel(esa_ref, eattr_ref, srcf_ref, dstf_ref,
                 W1k_ref, W2k_ref, W1v_ref, W2v_ref, Pk_ref, Pv_ref, Lt_ref,
                 ev_ref, expq_ref):
    esa = esa_ref[...]
    f32 = jnp.float32
    bf16 = jnp.bfloat16
    wk = jnp.maximum(esa @ W1k_ref[...], 0.0)
    wk = jnp.maximum(jnp.dot(wk, W2k_ref[...], preferred_element_type=f32), 0.0)
    esa_b = esa.astype(bf16)
    wv = jnp.maximum(jnp.dot(esa_b, W1v_ref[...].astype(bf16),
                             preferred_element_type=f32), 0.0)
    wv = jnp.maximum(jnp.dot(wv.astype(bf16), W2v_ref[...].astype(bf16),
                             preferred_element_type=f32), 0.0)
    wv_b = wv.astype(bf16)
    eattr = eattr_ref[...]
    srcf = srcf_ref[...]
    tk = None
    tv = None
    for v in range(DE):
        av = eattr[:, v:v + 1]
        pk = jnp.dot(wk, Pk_ref[v], preferred_element_type=f32)
        pv = jnp.dot(wv_b, Pv_ref[v].astype(bf16), preferred_element_type=f32)
        tk = pk * av if tk is None else tk + pk * av
        tv = pv * av if tv is None else tv + pv * av
    ek = srcf * tk * (1.0 / 16.0)
    ev = srcf * tv * (1.0 / 16.0)
    ev_ref[...] = ev
    K2 = jnp.dot(ek, Lt_ref[...], preferred_element_type=f32)
    dstf = dstf_ref[...]
    cols = []
    for h in range(H):
        lg = jnp.sum(dstf * K2[:, h * D:(h + 1) * D], axis=1, keepdims=True)
        cols.append(jnp.exp(lg * (0.25 / D)))
    expq_ref[...] = jnp.concatenate(cols, axis=1)


def _edge_stage(esa, eattr, srcf, dstf, W1k, W2k, W1v, W2v, Pk_t, Pv_t, Lt2):
    nb = E // BE
    full = lambda shape: pl.BlockSpec(shape, lambda i: (0,) * len(shape))
    return pl.pallas_call(
        _edge_kernel,
        grid=(nb,),
        in_specs=[
            pl.BlockSpec((BE, DS), lambda i: (i, 0)),
            pl.BlockSpec((BE, DE), lambda i: (i, 0)),
            pl.BlockSpec((BE, D), lambda i: (i, 0)),
            pl.BlockSpec((BE, D), lambda i: (i, 0)),
            full((DS, NW)), full((NW, NW)), full((DS, NW)), full((NW, NW)),
            full((DE, NW, D)), full((DE, NW, D)), full((D, H * D)),
        ],
        out_specs=[
            pl.BlockSpec((BE, D), lambda i: (i, 0)),
            pl.BlockSpec((BE, H), lambda i: (i, 0)),
        ],
        out_shape=[
            jax.ShapeDtypeStruct((E, D), jnp.float32),
            jax.ShapeDtypeStruct((E, H), jnp.float32),
        ],
        compiler_params=pltpu.CompilerParams(
            dimension_semantics=("arbitrary",)),
    )(esa, eattr, srcf, dstf, W1k, W2k, W1v, W2v, Pk_t, Pv_t, Lt2)


def _mul_kernel(ev_ref, scale_ref, B_ref, evw_ref):
    sc = jnp.dot(scale_ref[...], B_ref[...],
                 preferred_element_type=jnp.float32)
    evw_ref[...] = ev_ref[...] * sc


def _mul_stage(ev, scale2, Bmat):
    nb = E // BE
    return pl.pallas_call(
        _mul_kernel,
        grid=(nb,),
        in_specs=[
            pl.BlockSpec((BE, D), lambda i: (i, 0)),
            pl.BlockSpec((BE, H), lambda i: (i, 0)),
            pl.BlockSpec((H, D), lambda i: (0, 0)),
        ],
        out_specs=pl.BlockSpec((BE, D), lambda i: (i, 0)),
        out_shape=jax.ShapeDtypeStruct((E, D), jnp.float32),
        compiler_params=pltpu.CompilerParams(
            dimension_semantics=("arbitrary",)),
    )(ev, scale2, Bmat)


def _out_kernel(p_ref, w_ref, o_ref):
    o_ref[...] = jnp.dot(p_ref[0], w_ref[...],
                         preferred_element_type=jnp.float32)


def _out_stage(out_part, L_out_s):
    bn = 1000
    return pl.pallas_call(
        _out_kernel,
        grid=(N // bn,),
        in_specs=[pl.BlockSpec((1, bn, D), lambda i: (i // 5, i % 5, 0)),
                  pl.BlockSpec((D, D), lambda i: (0, 0))],
        out_specs=pl.BlockSpec((bn, D), lambda i: (i, 0)),
        out_shape=jax.ShapeDtypeStruct((N, D), jnp.float32),
    )(out_part, L_out_s)


def kernel(edge_src, edge_dst, edge_scalar_attr, edge_weight_cutoff, edge_attr,
           node_feat, Wk1, Wk2, Pk, Wv1, Wv2, Pv, L_logit, L_out):
    W1k = Wk1 * (1.0 / jnp.sqrt(DS))
    W2k = Wk2 * (1.0 / jnp.sqrt(NW))
    W1v = Wv1 * (1.0 / jnp.sqrt(DS))
    W2v = Wv2 * (1.0 / jnp.sqrt(NW))
    Pk_t = jnp.transpose(Pk, (2, 0, 1))
    Pv_t = jnp.transpose(Pv, (2, 0, 1))
    Lt2 = jnp.transpose(L_logit, (1, 2, 0)).reshape(D, H * D)
    L_out_s = L_out * (1.0 / jnp.sqrt(D))

    srcf, dstf = _kg_stage(edge_src, edge_dst, node_feat)

    ev, expq = _edge_stage(edge_scalar_attr, edge_attr, srcf, dstf,
                           W1k, W2k, W1v, W2v, Pk_t, Pv_t, Lt2)

    expq_flat = expq.reshape(E * H)

    dst_bits = lax.bitcast_convert_type(edge_dst, jnp.float32)
    s_part = _ks_stage(expq_flat, dst_bits)
    z_part, r_flat = _kz_stage(expq_flat, dst_bits, edge_weight_cutoff, s_part)
    scale_flat = _ksc_stage(r_flat, edge_weight_cutoff, dst_bits, z_part)

    Bmat = jnp.kron(jnp.eye(H, dtype=jnp.float32),
                    jnp.ones((1, D // H), jnp.float32))
    evw = _mul_stage(ev, scale_flat.reshape(E, H), Bmat)
    idx_lo = jnp.where(edge_dst < NHALF, edge_dst, NHALF)
    idx_hi = jnp.where(edge_dst >= NHALF, edge_dst - NHALF, NHALF)
    idx2 = jnp.concatenate([idx_lo[None, :], idx_hi[None, :]], axis=0)
    out_part = _kw_stage(evw, idx2)

    return _out_stage(out_part, L_out_s)


# speedup vs baseline: 1.1130x; 1.1130x over previous
import functools

import jax
import jax.numpy as jnp
from jax import lax
from jax.experimental import pallas as pl
from jax.experimental.pallas import tpu as pltpu
from jax.experimental.pallas import tpu_sc as plsc

N = 10000
E = 160000
D = 128
DE = 4
DS = 16
NW = 64
H = 4

NHP = 40960
NWORK = 32
BE = 2000

CE1 = 5000
NC1 = E // CE1
CPW1 = NC1 // NWORK
CE2 = 128
NC2 = E // CE2


def _mesh():
    return plsc.VectorSubcoreMesh(core_axis_name="c", subcore_axis_name="s")


def _lanes():
    return lax.iota(jnp.int32, 16)


def _zero_1d(ref, n):
    z = jnp.zeros((16,), jnp.float32)

    def body(i):
        ref[pl.ds(i * 16, 16)] = z

    plsc.parallel_loop(0, n // 16, unroll=8)(body)


def _rsqrt_quake(x):
    i = plsc.bitcast(x, jnp.int32)
    i = 0x5F3759DF - lax.shift_right_arithmetic(i, 1)
    y = plsc.bitcast(i, jnp.float32)
    for _ in range(3):
        y = y * (1.5 - 0.5 * x * y * y)
    return y


RBS = 8192
RSUB = RBS // 16


def _block_reduce(acc, shared, tmp, red, out_at_c, s):
    for b in range(NHP // RBS):
        pltpu.sync_copy(acc.at[pl.ds(b * RBS, RBS)], shared.at[s])
        plsc.subcore_barrier()
        off = s * RSUB
        _zero_1d(red, RSUB)
        for j in range(16):
            pltpu.sync_copy(shared.at[j, pl.ds(off, RSUB)], tmp)

            def radd(i):
                red[pl.ds(i * 16, 16)] = (red[pl.ds(i * 16, 16)]
                                          + tmp[pl.ds(i * 16, 16)])

            plsc.parallel_loop(0, RSUB // 16, unroll=8)(radd)
        pltpu.sync_copy(red, out_at_c.at[pl.ds(b * RBS + off, RSUB)])
        plsc.subcore_barrier()


def _kg_body(srci_hbm, dsti_hbm, nf_hbm, srcf_out, dstf_out,
             idxa, buf0, buf1, sem0, sem1):
    c = lax.axis_index("c")
    s = lax.axis_index("s")
    wid = c * 16 + s
    t = lax.shift_right_arithmetic(wid, 4)
    g = lax.bitwise_and(wid, 15)
    EPW = E // 16
    NF = EPW // CE2
    base = g * EPW

    def run(idx_hbm, out_hbm):
        pltpu.sync_copy(idx_hbm.at[pl.ds(base, EPW)], idxa)

        def pair(t2, _):
            oa = (2 * t2) * CE2
            ob = (2 * t2 + 1) * CE2
            a = pltpu.async_copy(nf_hbm.at[idxa.at[pl.ds(oa, CE2)]], buf0, sem0)
            b = pltpu.async_copy(nf_hbm.at[idxa.at[pl.ds(ob, CE2)]], buf1, sem1)
            a.wait()
            pltpu.sync_copy(buf0, out_hbm.at[pl.ds(base + oa, CE2), :])
            b.wait()
            pltpu.sync_copy(buf1, out_hbm.at[pl.ds(base + ob, CE2), :])
            return 0

        lax.fori_loop(0, NF // 2, pair, 0)
        o = NF * CE2
        pltpu.async_copy(nf_hbm.at[idxa.at[pl.ds(o, EPW - NF * CE2)]],
                         buf0.at[pl.ds(0, EPW - NF * CE2), :], sem0).wait()
        pltpu.sync_copy(buf0.at[pl.ds(0, EPW - NF * CE2), :],
                        out_hbm.at[pl.ds(base + o, EPW - NF * CE2), :])

    @pl.when(t == 0)
    def _():
        run(srci_hbm, srcf_out)

    @pl.when(t == 1)
    def _():
        run(dsti_hbm, dstf_out)


def _kg_stage(edge_src, edge_dst, node_feat):
    f = functools.partial(
        pl.kernel,
        mesh=_mesh(),
        compiler_params=pltpu.CompilerParams(needs_layout_passes=False),
        out_type=[
            jax.ShapeDtypeStruct((E, D), jnp.float32),
            jax.ShapeDtypeStruct((E, D), jnp.float32),
        ],
        scratch_types=[
            pltpu.VMEM((E // 16,), jnp.int32),
            pltpu.VMEM((CE2, D), jnp.float32),
            pltpu.VMEM((CE2, D), jnp.float32),
            pltpu.SemaphoreType.DMA,
            pltpu.SemaphoreType.DMA,
        ],
    )
    return f(_kg_body)(edge_src, edge_dst, node_feat)


def _ks_body(expq_hbm, dst_hbm, s_out, acc, dstv, qv, tmp, red, shared):
    c = lax.axis_index("c")
    s = lax.axis_index("s")
    wid = c * 16 + s
    lanes = _lanes()
    _zero_1d(acc, NHP)

    def chunk(ci, _):
        base = (wid * CPW1 + ci) * CE1
        pltpu.sync_copy(dst_hbm.at[pl.ds(base, CE1)], dstv)
        pltpu.sync_copy(expq_hbm.at[pl.ds(base * H, CE1 * H)], qv)

        def grp(g):
            p = g * 16 + lanes
            el = lax.shift_right_arithmetic(p, 2)
            h = lax.bitwise_and(p, 3)
            d = plsc.bitcast(plsc.load_gather(dstv, [el]), jnp.int32)
            idx = d * 4 + h
            val = qv[pl.ds(g * 16, 16)]
            plsc.addupdate_scatter(acc, [idx], val)

        plsc.parallel_loop(0, CE1 * H // 16, unroll=4)(grp)
        return 0

    lax.fori_loop(0, CPW1, chunk, 0)

    _block_reduce(acc, shared, tmp, red, s_out.at[c], s)


def _ks_stage(expq_flat, dst_flat):
    f = functools.partial(
        pl.kernel,
        mesh=_mesh(),
        compiler_params=pltpu.CompilerParams(needs_layout_passes=False),
        out_type=jax.ShapeDtypeStruct((2, NHP), jnp.float32),
        scratch_types=[
            pltpu.VMEM((NHP,), jnp.float32),
            pltpu.VMEM((CE1,), jnp.float32),
            pltpu.VMEM((CE1 * H,), jnp.float32),
            pltpu.VMEM((RSUB,), jnp.float32),
            pltpu.VMEM((RSUB,), jnp.float32),
            pltpu.VMEM_SHARED((16, RBS), jnp.float32),
        ],
    )
    return f(_ks_body)(expq_flat, dst_flat)


def _kz_body(expq_hbm, dst_hbm, cut_hbm, s_part, z_out, r_out,
             stab, zacc, dstv, qv, cutv, tmp, red, shared):
    c = lax.axis_index("c")
    s = lax.axis_index("s")
    wid = c * 16 + s
    lanes = _lanes()

    pltpu.sync_copy(s_part.at[0], stab)
    for b in range(NHP // RSUB):
        pltpu.sync_copy(s_part.at[1, pl.ds(b * RSUB, RSUB)], tmp)

        def badd(i):
            o = b * RSUB + i * 16
            stab[pl.ds(o, 16)] = stab[pl.ds(o, 16)] + tmp[pl.ds(i * 16, 16)]

        plsc.parallel_loop(0, RSUB // 16, unroll=8)(badd)

    _zero_1d(zacc, NHP)

    def chunk(ci, _):
        base = (wid * CPW1 + ci) * CE1
        pltpu.sync_copy(dst_hbm.at[pl.ds(base, CE1)], dstv)
        pltpu.sync_copy(expq_hbm.at[pl.ds(base * H, CE1 * H)], qv)
        pltpu.sync_copy(cut_hbm.at[pl.ds(base, CE1)], cutv)

        def grp(g):
            p = g * 16 + lanes
            el = lax.shift_right_arithmetic(p, 2)
            h = lax.bitwise_and(p, 3)
            d = plsc.bitcast(plsc.load_gather(dstv, [el]), jnp.int32)
            idx = d * 4 + h
            sv = plsc.load_gather(stab, [idx])
            q = qv[pl.ds(g * 16, 16)]
            r = q / sv
            qv[pl.ds(g * 16, 16)] = r
            r2 = r * r
            cu = plsc.load_gather(cutv, [el])
            plsc.addupdate_scatter(zacc, [idx], cu * r2 * r2)

        plsc.parallel_loop(0, CE1 * H // 16, unroll=4)(grp)
        pltpu.sync_copy(qv, r_out.at[pl.ds(base * H, CE1 * H)])
        return 0

    lax.fori_loop(0, CPW1, chunk, 0)

    _block_reduce(zacc, shared, tmp, red, z_out.at[c], s)


def _kz_stage(expq_flat, dst_flat, cut, s_part):
    f = functools.partial(
        pl.kernel,
        mesh=_mesh(),
        compiler_params=pltpu.CompilerParams(needs_layout_passes=False),
        out_type=[
            jax.ShapeDtypeStruct((2, NHP), jnp.float32),
            jax.ShapeDtypeStruct((E * H,), jnp.float32),
        ],
        scratch_types=[
            pltpu.VMEM((NHP,), jnp.float32),
            pltpu.VMEM((NHP,), jnp.float32),
            pltpu.VMEM((CE1,), jnp.float32),
            pltpu.VMEM((CE1 * H,), jnp.float32),
            pltpu.VMEM((CE1,), jnp.float32),
            pltpu.VMEM((RSUB,), jnp.float32),
            pltpu.VMEM((RSUB,), jnp.float32),
            pltpu.VMEM_SHARED((16, RBS), jnp.float32),
        ],
    )
    return f(_kz_body)(expq_flat, dst_flat, cut, s_part)


def _ksc_body(r_hbm, cut_hbm, dst_hbm, z_part, scale_out,
              ztab, tmp, rv, cutv, dstv):
    c = lax.axis_index("c")
    s = lax.axis_index("s")
    wid = c * 16 + s
    lanes = _lanes()

    pltpu.sync_copy(z_part.at[0], ztab)
    ZB = 4096
    for b in range(NHP // ZB):
        pltpu.sync_copy(z_part.at[1, pl.ds(b * ZB, ZB)], tmp)

        def badd(i):
            o = b * ZB + i * 16
            x = ztab[pl.ds(o, 16)] + tmp[pl.ds(i * 16, 16)]
            xc = jnp.where(x == 0.0, 1.0, x)
            ztab[pl.ds(o, 16)] = _rsqrt_quake(xc)

        plsc.parallel_loop(0, ZB // 16, unroll=4)(badd)

    def chunk(ci, _):
        base = (wid * CPW1 + ci) * CE1
        pltpu.sync_copy(r_hbm.at[pl.ds(base * H, CE1 * H)], rv)
        pltpu.sync_copy(cut_hbm.at[pl.ds(base, CE1)], cutv)
        pltpu.sync_copy(dst_hbm.at[pl.ds(base, CE1)], dstv)

        def grp(g):
            p = g * 16 + lanes
            el = lax.shift_right_arithmetic(p, 2)
            h = lax.bitwise_and(p, 3)
            d = plsc.bitcast(plsc.load_gather(dstv, [el]), jnp.int32)
            iz = plsc.load_gather(ztab, [d * 4 + h])
            cu = plsc.load_gather(cutv, [el])
            sq = cu * _rsqrt_quake(jnp.where(cu == 0.0, 1.0, cu))
            rr = rv[pl.ds(g * 16, 16)]
            rv[pl.ds(g * 16, 16)] = rr * rr * sq * iz

        plsc.parallel_loop(0, CE1 * H // 16, unroll=4)(grp)
        pltpu.sync_copy(rv, scale_out.at[pl.ds(base * H, CE1 * H)])
        return 0

    lax.fori_loop(0, CPW1, chunk, 0)


def _ksc_stage(r_flat, cut, dst_bits, z_part):
    f = functools.partial(
        pl.kernel,
        mesh=_mesh(),
        compiler_params=pltpu.CompilerParams(needs_layout_passes=False),
        out_type=jax.ShapeDtypeStruct((E * H,), jnp.float32),
        scratch_types=[
            pltpu.VMEM((NHP,), jnp.float32),
            pltpu.VMEM((4096,), jnp.float32),
            pltpu.VMEM((CE1 * H,), jnp.float32),
            pltpu.VMEM((CE1,), jnp.float32),
            pltpu.VMEM((CE1,), jnp.float32),
        ],
    )
    return f(_ksc_body)(r_flat, cut, dst_bits, z_part)


NHALF = 5000
NROWS = 5120


def _kw_body(evw_hbm, idx2_hbm, out_part, evv, idxv, evv1, idxv1,
             sem0, sem1, nodeacc):
    c = lax.axis_index("c")
    s = lax.axis_index("s")
    zv = jnp.zeros((16,), jnp.float32)

    def zb(t):
        evv[lax.shift_right_arithmetic(t, 3),
            pl.ds(lax.bitwise_and(t, 7) * 16, 16)] = zv

    plsc.parallel_loop(0, 128 * 8, unroll=8)(zb)
    for k, nr in ((0, 128), (1, 128), (2, 64)):
        pltpu.sync_copy(evv.at[pl.ds(0, nr), :],
                        nodeacc.at[pl.ds(s * 320 + k * 128, nr), :])
    plsc.subcore_barrier()

    def pair(t, _):
        base_a = ((2 * t) * 16 + s) * CE2
        base_b = ((2 * t + 1) * 16 + s) * CE2
        a0 = pltpu.async_copy(evw_hbm.at[pl.ds(base_a, CE2), :], evv, sem0)
        a1 = pltpu.async_copy(idx2_hbm.at[c, pl.ds(base_a, CE2)], idxv, sem0)
        b0 = pltpu.async_copy(evw_hbm.at[pl.ds(base_b, CE2), :], evv1, sem1)
        b1 = pltpu.async_copy(idx2_hbm.at[c, pl.ds(base_b, CE2)], idxv1, sem1)
        a0.wait()
        a1.wait()
        pltpu.sync_copy(evv, nodeacc.at[idxv], add=True)
        b0.wait()
        b1.wait()
        pltpu.sync_copy(evv1, nodeacc.at[idxv1], add=True)
        return 0

    lax.fori_loop(0, NC2 // 32, pair, 0)

    @pl.when(s < NC2 % 16)
    def _():
        base = ((NC2 // 16) * 16 + s) * CE2
        pltpu.sync_copy(evw_hbm.at[pl.ds(base, CE2), :], evv)
        pltpu.sync_copy(idx2_hbm.at[c, pl.ds(base, CE2)], idxv)
        pltpu.sync_copy(evv, nodeacc.at[idxv], add=True)

    plsc.subcore_barrier()
    for k, nr in ((0, 128), (1, 128), (2, 64)):
        o = s * 320 + k * 128
        pltpu.sync_copy(nodeacc.at[pl.ds(o, nr), :],
                        out_part.at[c, pl.ds(o, nr), :])


def _kw_stage(evw, idx2):
    f = functools.partial(
        pl.kernel,
        mesh=_mesh(),
        compiler_params=pltpu.CompilerParams(needs_layout_passes=False),
        out_type=jax.ShapeDtypeStruct((2, NROWS, D), jnp.float32),
        scratch_types=[
            pltpu.VMEM((CE2, D), jnp.float32),
            pltpu.VMEM((CE2,), jnp.int32),
            pltpu.VMEM((CE2, D), jnp.float32),
            pltpu.VMEM((CE2,), jnp.int32),
            pltpu.SemaphoreType.DMA,
            pltpu.SemaphoreType.DMA,
            pltpu.VMEM_SHARED((NROWS, D), jnp.float32),
        ],
    )
    return f(_kw_body)(evw, idx2)


def _edge_kernel(esa_ref, eattr_ref, srcf_ref, dstf_ref,
                 W1k_ref, W2k_ref, W1v_ref, W2v_ref, Pk_ref, Pv_ref, Lt_ref,
                 ev_ref, expq_ref):
    esa = esa_ref[...]
    f32 = jnp.float32
    wk = jnp.maximum(esa @ W1k_ref[...], 0.0)
    wk = jnp.maximum(jnp.dot(wk, W2k_ref[...], preferred_element_type=f32), 0.0)
    wv = jnp.maximum(esa @ W1v_ref[...], 0.0)
    wv = jnp.maximum(jnp.dot(wv, W2v_ref[...], preferred_element_type=f32), 0.0)
    eattr = eattr_ref[...]
    srcf = srcf_ref[...]
    tk = None
    tv = None
    for v in range(DE):
        av = eattr[:, v:v + 1]
        pk = jnp.dot(wk, Pk_ref[v], preferred_element_type=f32)
        pv = jnp.dot(wv, Pv_ref[v], preferred_element_type=f32)
        tk = pk * av if tk is None else tk + pk * av
        tv = pv * av if tv is None else tv + pv * av
    ek = srcf * tk * (1.0 / 16.0)
    ev = srcf * tv * (1.0 / 16.0)
    ev_ref[...] = ev
    K2 = jnp.dot(ek, Lt_ref[...], preferred_element_type=f32)
    dstf = dstf_ref[...]
    cols = []
    for h in range(H):
        lg = jnp.sum(dstf * K2[:, h * D:(h + 1) * D], axis=1, keepdims=True)
        cols.append(jnp.exp(lg * (0.25 / D)))
    expq_ref[...] = jnp.concatenate(cols, axis=1)


def _edge_stage(esa, eattr, srcf, dstf, W1k, W2k, W1v, W2v, Pk_t, Pv_t, Lt2):
    nb = E // BE
    full = lambda shape: pl.BlockSpec(shape, lambda i: (0,) * len(shape))
    return pl.pallas_call(
        _edge_kernel,
        grid=(nb,),
        in_specs=[
            pl.BlockSpec((BE, DS), lambda i: (i, 0)),
            pl.BlockSpec((BE, DE), lambda i: (i, 0)),
            pl.BlockSpec((BE, D), lambda i: (i, 0)),
            pl.BlockSpec((BE, D), lambda i: (i, 0)),
            full((DS, NW)), full((NW, NW)), full((DS, NW)), full((NW, NW)),
            full((DE, NW, D)), full((DE, NW, D)), full((D, H * D)),
        ],
        out_specs=[
            pl.BlockSpec((BE, D), lambda i: (i, 0)),
            pl.BlockSpec((BE, H), lambda i: (i, 0)),
        ],
        out_shape=[
            jax.ShapeDtypeStruct((E, D), jnp.float32),
            jax.ShapeDtypeStruct((E, H), jnp.float32),
        ],
        compiler_params=pltpu.CompilerParams(
            dimension_semantics=("arbitrary",)),
    )(esa, eattr, srcf, dstf, W1k, W2k, W1v, W2v, Pk_t, Pv_t, Lt2)


def _mul_kernel(ev_ref, scale_ref, B_ref, evw_ref):
    sc = jnp.dot(scale_ref[...], B_ref[...],
                 preferred_element_type=jnp.float32)
    evw_ref[...] = ev_ref[...] * sc


def _mul_stage(ev, scale2, Bmat):
    nb = E // BE
    return pl.pallas_call(
        _mul_kernel,
        grid=(nb,),
        in_specs=[
            pl.BlockSpec((BE, D), lambda i: (i, 0)),
            pl.BlockSpec((BE, H), lambda i: (i, 0)),
            pl.BlockSpec((H, D), lambda i: (0, 0)),
        ],
        out_specs=pl.BlockSpec((BE, D), lambda i: (i, 0)),
        out_shape=jax.ShapeDtypeStruct((E, D), jnp.float32),
        compiler_params=pltpu.CompilerParams(
            dimension_semantics=("arbitrary",)),
    )(ev, scale2, Bmat)


def _out_kernel(p_ref, w_ref, o_ref):
    o_ref[...] = jnp.dot(p_ref[0], w_ref[...],
                         preferred_element_type=jnp.float32)


def _out_stage(out_part, L_out_s):
    bn = 1000
    return pl.pallas_call(
        _out_kernel,
        grid=(N // bn,),
        in_specs=[pl.BlockSpec((1, bn, D), lambda i: (i // 5, i % 5, 0)),
                  pl.BlockSpec((D, D), lambda i: (0, 0))],
        out_specs=pl.BlockSpec((bn, D), lambda i: (i, 0)),
        out_shape=jax.ShapeDtypeStruct((N, D), jnp.float32),
    )(out_part, L_out_s)


def kernel(edge_src, edge_dst, edge_scalar_attr, edge_weight_cutoff, edge_attr,
           node_feat, Wk1, Wk2, Pk, Wv1, Wv2, Pv, L_logit, L_out):
    W1k = Wk1 * (1.0 / jnp.sqrt(DS))
    W2k = Wk2 * (1.0 / jnp.sqrt(NW))
    W1v = Wv1 * (1.0 / jnp.sqrt(DS))
    W2v = Wv2 * (1.0 / jnp.sqrt(NW))
    Pk_t = jnp.transpose(Pk, (2, 0, 1))
    Pv_t = jnp.transpose(Pv, (2, 0, 1))
    Lt2 = jnp.transpose(L_logit, (1, 2, 0)).reshape(D, H * D)
    L_out_s = L_out * (1.0 / jnp.sqrt(D))

    srcf, dstf = _kg_stage(edge_src, edge_dst, node_feat)

    ev, expq = _edge_stage(edge_scalar_attr, edge_attr, srcf, dstf,
                           W1k, W2k, W1v, W2v, Pk_t, Pv_t, Lt2)

    expq_flat = expq.reshape(E * H)

    dst_bits = lax.bitcast_convert_type(edge_dst, jnp.float32)
    s_part = _ks_stage(expq_flat, dst_bits)
    z_part, r_flat = _kz_stage(expq_flat, dst_bits, edge_weight_cutoff, s_part)
    scale_flat = _ksc_stage(r_flat, edge_weight_cutoff, dst_bits, z_part)

    Bmat = jnp.kron(jnp.eye(H, dtype=jnp.float32),
                    jnp.ones((1, D // H), jnp.float32))
    evw = _mul_stage(ev, scale_flat.reshape(E, H), Bmat)
    idx_lo = jnp.where(edge_dst < NHALF, edge_dst, NHALF)
    idx_hi = jnp.where(edge_dst >= NHALF, edge_dst - NHALF, NHALF)
    idx2 = jnp.concatenate([idx_lo[None, :], idx_hi[None, :]], axis=0)
    out_part = _kw_stage(evw, idx2)

    return _out_stage(out_part, L_out_s)


# speedup vs baseline: 1.1165x; 1.0032x over previous
import functools

import jax
import jax.numpy as jnp
from jax import lax
from jax.experimental import pallas as pl
from jax.experimental.pallas import tpu as pltpu
from jax.experimental.pallas import tpu_sc as plsc

N = 10000
E = 160000
D = 128
DE = 4
DS = 16
NW = 64
H = 4

NHP = 40960
NWORK = 32
BE = 2000

CE1 = 5000
NC1 = E // CE1
CPW1 = NC1 // NWORK
CE2 = 128
NC2 = E // CE2


def _mesh():
    return plsc.VectorSubcoreMesh(core_axis_name="c", subcore_axis_name="s")


def _lanes():
    return lax.iota(jnp.int32, 16)


def _zero_1d(ref, n):
    z = jnp.zeros((16,), jnp.float32)

    def body(i):
        ref[pl.ds(i * 16, 16)] = z

    plsc.parallel_loop(0, n // 16, unroll=8)(body)


def _rsqrt_quake(x):
    i = plsc.bitcast(x, jnp.int32)
    i = 0x5F3759DF - lax.shift_right_arithmetic(i, 1)
    y = plsc.bitcast(i, jnp.float32)
    for _ in range(3):
        y = y * (1.5 - 0.5 * x * y * y)
    return y


RBS = 8192
RSUB = RBS // 16


def _block_reduce(acc, shared, tmp, red, out_at_c, s):
    for b in range(NHP // RBS):
        pltpu.sync_copy(acc.at[pl.ds(b * RBS, RBS)], shared.at[s])
        plsc.subcore_barrier()
        off = s * RSUB
        _zero_1d(red, RSUB)
        for j in range(16):
            pltpu.sync_copy(shared.at[j, pl.ds(off, RSUB)], tmp)

            def radd(i):
                red[pl.ds(i * 16, 16)] = (red[pl.ds(i * 16, 16)]
                                          + tmp[pl.ds(i * 16, 16)])

            plsc.parallel_loop(0, RSUB // 16, unroll=8)(radd)
        pltpu.sync_copy(red, out_at_c.at[pl.ds(b * RBS + off, RSUB)])
        plsc.subcore_barrier()


def _kg_body(srci_hbm, dsti_hbm, nf_hbm, srcf_out, dstf_out,
             idxa, buf0, buf1, sem0, sem1):
    c = lax.axis_index("c")
    s = lax.axis_index("s")
    wid = c * 16 + s
    t = lax.shift_right_arithmetic(wid, 4)
    g = lax.bitwise_and(wid, 15)
    EPW = E // 16
    NF = EPW // CE2
    base = g * EPW

    def run(idx_hbm, out_hbm):
        pltpu.sync_copy(idx_hbm.at[pl.ds(base, EPW)], idxa)

        def pair(t2, _):
            oa = (2 * t2) * CE2
            ob = (2 * t2 + 1) * CE2
            a = pltpu.async_copy(nf_hbm.at[idxa.at[pl.ds(oa, CE2)]], buf0, sem0)
            b = pltpu.async_copy(nf_hbm.at[idxa.at[pl.ds(ob, CE2)]], buf1, sem1)
            a.wait()
            pltpu.sync_copy(buf0, out_hbm.at[pl.ds(base + oa, CE2), :])
            b.wait()
            pltpu.sync_copy(buf1, out_hbm.at[pl.ds(base + ob, CE2), :])
            return 0

        lax.fori_loop(0, NF // 2, pair, 0)
        o = NF * CE2
        pltpu.async_copy(nf_hbm.at[idxa.at[pl.ds(o, EPW - NF * CE2)]],
                         buf0.at[pl.ds(0, EPW - NF * CE2), :], sem0).wait()
        pltpu.sync_copy(buf0.at[pl.ds(0, EPW - NF * CE2), :],
                        out_hbm.at[pl.ds(base + o, EPW - NF * CE2), :])

    @pl.when(t == 0)
    def _():
        run(srci_hbm, srcf_out)

    @pl.when(t == 1)
    def _():
        run(dsti_hbm, dstf_out)


def _kg_stage(edge_src, edge_dst, node_feat):
    f = functools.partial(
        pl.kernel,
        mesh=_mesh(),
        compiler_params=pltpu.CompilerParams(needs_layout_passes=False),
        out_type=[
            jax.ShapeDtypeStruct((E, D), jnp.float32),
            jax.ShapeDtypeStruct((E, D), jnp.float32),
        ],
        scratch_types=[
            pltpu.VMEM((E // 16,), jnp.int32),
            pltpu.VMEM((CE2, D), jnp.float32),
            pltpu.VMEM((CE2, D), jnp.float32),
            pltpu.SemaphoreType.DMA,
            pltpu.SemaphoreType.DMA,
        ],
    )
    return f(_kg_body)(edge_src, edge_dst, node_feat)


def _ks_body(expq_hbm, dst_hbm, s_out, acc, dstv, qv, tmp, red, shared):
    c = lax.axis_index("c")
    s = lax.axis_index("s")
    wid = c * 16 + s
    lanes = _lanes()
    _zero_1d(acc, NHP)

    def chunk(ci, _):
        base = (wid * CPW1 + ci) * CE1
        pltpu.sync_copy(dst_hbm.at[pl.ds(base, CE1)], dstv)
        pltpu.sync_copy(expq_hbm.at[pl.ds(base * H, CE1 * H)], qv)

        def grp(g):
            p = g * 16 + lanes
            el = lax.shift_right_arithmetic(p, 2)
            h = lax.bitwise_and(p, 3)
            d = plsc.bitcast(plsc.load_gather(dstv, [el]), jnp.int32)
            idx = d * 4 + h
            val = qv[pl.ds(g * 16, 16)]
            plsc.addupdate_scatter(acc, [idx], val)

        plsc.parallel_loop(0, CE1 * H // 16, unroll=8)(grp)
        return 0

    lax.fori_loop(0, CPW1, chunk, 0)

    _block_reduce(acc, shared, tmp, red, s_out.at[c], s)


def _ks_stage(expq_flat, dst_flat):
    f = functools.partial(
        pl.kernel,
        mesh=_mesh(),
        compiler_params=pltpu.CompilerParams(needs_layout_passes=False),
        out_type=jax.ShapeDtypeStruct((2, NHP), jnp.float32),
        scratch_types=[
            pltpu.VMEM((NHP,), jnp.float32),
            pltpu.VMEM((CE1,), jnp.float32),
            pltpu.VMEM((CE1 * H,), jnp.float32),
            pltpu.VMEM((RSUB,), jnp.float32),
            pltpu.VMEM((RSUB,), jnp.float32),
            pltpu.VMEM_SHARED((16, RBS), jnp.float32),
        ],
    )
    return f(_ks_body)(expq_flat, dst_flat)


def _kz_body(expq_hbm, dst_hbm, cut_hbm, s_part, z_out, r_out,
             stab, zacc, dstv, qv, cutv, tmp, red, shared):
    c = lax.axis_index("c")
    s = lax.axis_index("s")
    wid = c * 16 + s
    lanes = _lanes()

    pltpu.sync_copy(s_part.at[0], stab)
    for b in range(NHP // RSUB):
        pltpu.sync_copy(s_part.at[1, pl.ds(b * RSUB, RSUB)], tmp)

        def badd(i):
            o = b * RSUB + i * 16
            stab[pl.ds(o, 16)] = stab[pl.ds(o, 16)] + tmp[pl.ds(i * 16, 16)]

        plsc.parallel_loop(0, RSUB // 16, unroll=8)(badd)

    _zero_1d(zacc, NHP)

    def chunk(ci, _):
        base = (wid * CPW1 + ci) * CE1
        pltpu.sync_copy(dst_hbm.at[pl.ds(base, CE1)], dstv)
        pltpu.sync_copy(expq_hbm.at[pl.ds(base * H, CE1 * H)], qv)
        pltpu.sync_copy(cut_hbm.at[pl.ds(base, CE1)], cutv)

        def grp(g):
            p = g * 16 + lanes
            el = lax.shift_right_arithmetic(p, 2)
            h = lax.bitwise_and(p, 3)
            d = plsc.bitcast(plsc.load_gather(dstv, [el]), jnp.int32)
            idx = d * 4 + h
            sv = plsc.load_gather(stab, [idx])
            q = qv[pl.ds(g * 16, 16)]
            r = q / sv
            qv[pl.ds(g * 16, 16)] = r
            r2 = r * r
            cu = plsc.load_gather(cutv, [el])
            plsc.addupdate_scatter(zacc, [idx], cu * r2 * r2)

        plsc.parallel_loop(0, CE1 * H // 16, unroll=8)(grp)
        pltpu.sync_copy(qv, r_out.at[pl.ds(base * H, CE1 * H)])
        return 0

    lax.fori_loop(0, CPW1, chunk, 0)

    _block_reduce(zacc, shared, tmp, red, z_out.at[c], s)


def _kz_stage(expq_flat, dst_flat, cut, s_part):
    f = functools.partial(
        pl.kernel,
        mesh=_mesh(),
        compiler_params=pltpu.CompilerParams(needs_layout_passes=False),
        out_type=[
            jax.ShapeDtypeStruct((2, NHP), jnp.float32),
            jax.ShapeDtypeStruct((E * H,), jnp.float32),
        ],
        scratch_types=[
            pltpu.VMEM((NHP,), jnp.float32),
            pltpu.VMEM((NHP,), jnp.float32),
            pltpu.VMEM((CE1,), jnp.float32),
            pltpu.VMEM((CE1 * H,), jnp.float32),
            pltpu.VMEM((CE1,), jnp.float32),
            pltpu.VMEM((RSUB,), jnp.float32),
            pltpu.VMEM((RSUB,), jnp.float32),
            pltpu.VMEM_SHARED((16, RBS), jnp.float32),
        ],
    )
    return f(_kz_body)(expq_flat, dst_flat, cut, s_part)


def _ksc_body(r_hbm, cut_hbm, dst_hbm, z_part, scale_out,
              ztab, tmp, rv, cutv, dstv):
    c = lax.axis_index("c")
    s = lax.axis_index("s")
    wid = c * 16 + s
    lanes = _lanes()

    pltpu.sync_copy(z_part.at[0], ztab)
    ZB = 4096
    for b in range(NHP // ZB):
        pltpu.sync_copy(z_part.at[1, pl.ds(b * ZB, ZB)], tmp)

        def badd(i):
            o = b * ZB + i * 16
            x = ztab[pl.ds(o, 16)] + tmp[pl.ds(i * 16, 16)]
            xc = jnp.where(x == 0.0, 1.0, x)
            ztab[pl.ds(o, 16)] = _rsqrt_quake(xc)

        plsc.parallel_loop(0, ZB // 16, unroll=4)(badd)

    def chunk(ci, _):
        base = (wid * CPW1 + ci) * CE1
        pltpu.sync_copy(r_hbm.at[pl.ds(base * H, CE1 * H)], rv)
        pltpu.sync_copy(cut_hbm.at[pl.ds(base, CE1)], cutv)
        pltpu.sync_copy(dst_hbm.at[pl.ds(base, CE1)], dstv)

        def grp(g):
            p = g * 16 + lanes
            el = lax.shift_right_arithmetic(p, 2)
            h = lax.bitwise_and(p, 3)
            d = plsc.bitcast(plsc.load_gather(dstv, [el]), jnp.int32)
            iz = plsc.load_gather(ztab, [d * 4 + h])
            cu = plsc.load_gather(cutv, [el])
            sq = cu * _rsqrt_quake(jnp.where(cu == 0.0, 1.0, cu))
            rr = rv[pl.ds(g * 16, 16)]
            rv[pl.ds(g * 16, 16)] = rr * rr * sq * iz

        plsc.parallel_loop(0, CE1 * H // 16, unroll=8)(grp)
        pltpu.sync_copy(rv, scale_out.at[pl.ds(base * H, CE1 * H)])
        return 0

    lax.fori_loop(0, CPW1, chunk, 0)


def _ksc_stage(r_flat, cut, dst_bits, z_part):
    f = functools.partial(
        pl.kernel,
        mesh=_mesh(),
        compiler_params=pltpu.CompilerParams(needs_layout_passes=False),
        out_type=jax.ShapeDtypeStruct((E * H,), jnp.float32),
        scratch_types=[
            pltpu.VMEM((NHP,), jnp.float32),
            pltpu.VMEM((4096,), jnp.float32),
            pltpu.VMEM((CE1 * H,), jnp.float32),
            pltpu.VMEM((CE1,), jnp.float32),
            pltpu.VMEM((CE1,), jnp.float32),
        ],
    )
    return f(_ksc_body)(r_flat, cut, dst_bits, z_part)


NHALF = 5000
NROWS = 5120


def _kw_body(evw_hbm, idx2_hbm, out_part, evv, idxv, evv1, idxv1,
             sem0, sem1, nodeacc):
    c = lax.axis_index("c")
    s = lax.axis_index("s")
    zv = jnp.zeros((16,), jnp.float32)

    def zb(t):
        evv[lax.shift_right_arithmetic(t, 3),
            pl.ds(lax.bitwise_and(t, 7) * 16, 16)] = zv

    plsc.parallel_loop(0, 128 * 8, unroll=8)(zb)
    for k, nr in ((0, 128), (1, 128), (2, 64)):
        pltpu.sync_copy(evv.at[pl.ds(0, nr), :],
                        nodeacc.at[pl.ds(s * 320 + k * 128, nr), :])
    plsc.subcore_barrier()

    def pair(t, _):
        base_a = ((2 * t) * 16 + s) * CE2
        base_b = ((2 * t + 1) * 16 + s) * CE2
        a0 = pltpu.async_copy(evw_hbm.at[pl.ds(base_a, CE2), :], evv, sem0)
        a1 = pltpu.async_copy(idx2_hbm.at[c, pl.ds(base_a, CE2)], idxv, sem0)
        b0 = pltpu.async_copy(evw_hbm.at[pl.ds(base_b, CE2), :], evv1, sem1)
        b1 = pltpu.async_copy(idx2_hbm.at[c, pl.ds(base_b, CE2)], idxv1, sem1)
        a0.wait()
        a1.wait()
        pltpu.sync_copy(evv, nodeacc.at[idxv], add=True)
        b0.wait()
        b1.wait()
        pltpu.sync_copy(evv1, nodeacc.at[idxv1], add=True)
        return 0

    lax.fori_loop(0, NC2 // 32, pair, 0)

    @pl.when(s < NC2 % 16)
    def _():
        base = ((NC2 // 16) * 16 + s) * CE2
        pltpu.sync_copy(evw_hbm.at[pl.ds(base, CE2), :], evv)
        pltpu.sync_copy(idx2_hbm.at[c, pl.ds(base, CE2)], idxv)
        pltpu.sync_copy(evv, nodeacc.at[idxv], add=True)

    plsc.subcore_barrier()
    for k, nr in ((0, 128), (1, 128), (2, 64)):
        o = s * 320 + k * 128
        pltpu.sync_copy(nodeacc.at[pl.ds(o, nr), :],
                        out_part.at[c, pl.ds(o, nr), :])


def _kw_stage(evw, idx2):
    f = functools.partial(
        pl.kernel,
        mesh=_mesh(),
        compiler_params=pltpu.CompilerParams(needs_layout_passes=False),
        out_type=jax.ShapeDtypeStruct((2, NROWS, D), jnp.float32),
        scratch_types=[
            pltpu.VMEM((CE2, D), jnp.float32),
            pltpu.VMEM((CE2,), jnp.int32),
            pltpu.VMEM((CE2, D), jnp.float32),
            pltpu.VMEM((CE2,), jnp.int32),
            pltpu.SemaphoreType.DMA,
            pltpu.SemaphoreType.DMA,
            pltpu.VMEM_SHARED((NROWS, D), jnp.float32),
        ],
    )
    return f(_kw_body)(evw, idx2)


def _edge_kernel(esa_ref, eattr_ref, srcf_ref, dstf_ref,
                 W1k_ref, W2k_ref, W1v_ref, W2v_ref, Pk_ref, Pv_ref, Lt_ref,
                 ev_ref, expq_ref):
    esa = esa_ref[...]
    f32 = jnp.float32
    wk = jnp.maximum(esa @ W1k_ref[...], 0.0)
    wk = jnp.maximum(jnp.dot(wk, W2k_ref[...], preferred_element_type=f32), 0.0)
    wv = jnp.maximum(esa @ W1v_ref[...], 0.0)
    wv = jnp.maximum(jnp.dot(wv, W2v_ref[...], preferred_element_type=f32), 0.0)
    eattr = eattr_ref[...]
    srcf = srcf_ref[...]
    tk = None
    tv = None
    for v in range(DE):
        av = eattr[:, v:v + 1]
        pk = jnp.dot(wk, Pk_ref[v], preferred_element_type=f32)
        pv = jnp.dot(wv, Pv_ref[v], preferred_element_type=f32)
        tk = pk * av if tk is None else tk + pk * av
        tv = pv * av if tv is None else tv + pv * av
    ek = srcf * tk * (1.0 / 16.0)
    ev = srcf * tv * (1.0 / 16.0)
    ev_ref[...] = ev
    K2 = jnp.dot(ek, Lt_ref[...], preferred_element_type=f32)
    dstf = dstf_ref[...]
    cols = []
    for h in range(H):
        lg = jnp.sum(dstf * K2[:, h * D:(h + 1) * D], axis=1, keepdims=True)
        cols.append(jnp.exp(lg * (0.25 / D)))
    expq_ref[...] = jnp.concatenate(cols, axis=1)


def _edge_stage(esa, eattr, srcf, dstf, W1k, W2k, W1v, W2v, Pk_t, Pv_t, Lt2):
    nb = E // BE
    full = lambda shape: pl.BlockSpec(shape, lambda i: (0,) * len(shape))
    return pl.pallas_call(
        _edge_kernel,
        grid=(nb,),
        in_specs=[
            pl.BlockSpec((BE, DS), lambda i: (i, 0)),
            pl.BlockSpec((BE, DE), lambda i: (i, 0)),
            pl.BlockSpec((BE, D), lambda i: (i, 0)),
            pl.BlockSpec((BE, D), lambda i: (i, 0)),
            full((DS, NW)), full((NW, NW)), full((DS, NW)), full((NW, NW)),
            full((DE, NW, D)), full((DE, NW, D)), full((D, H * D)),
        ],
        out_specs=[
            pl.BlockSpec((BE, D), lambda i: (i, 0)),
            pl.BlockSpec((BE, H), lambda i: (i, 0)),
        ],
        out_shape=[
            jax.ShapeDtypeStruct((E, D), jnp.float32),
            jax.ShapeDtypeStruct((E, H), jnp.float32),
        ],
        compiler_params=pltpu.CompilerParams(
            dimension_semantics=("arbitrary",)),
    )(esa, eattr, srcf, dstf, W1k, W2k, W1v, W2v, Pk_t, Pv_t, Lt2)


def _mul_kernel(ev_ref, scale_ref, B_ref, evw_ref):
    sc = jnp.dot(scale_ref[...], B_ref[...],
                 preferred_element_type=jnp.float32)
    evw_ref[...] = ev_ref[...] * sc


def _mul_stage(ev, scale2, Bmat):
    nb = E // BE
    return pl.pallas_call(
        _mul_kernel,
        grid=(nb,),
        in_specs=[
            pl.BlockSpec((BE, D), lambda i: (i, 0)),
            pl.BlockSpec((BE, H), lambda i: (i, 0)),
            pl.BlockSpec((H, D), lambda i: (0, 0)),
        ],
        out_specs=pl.BlockSpec((BE, D), lambda i: (i, 0)),
        out_shape=jax.ShapeDtypeStruct((E, D), jnp.float32),
        compiler_params=pltpu.CompilerParams(
            dimension_semantics=("arbitrary",)),
    )(ev, scale2, Bmat)


def _out_kernel(p_ref, w_ref, o_ref):
    o_ref[...] = jnp.dot(p_ref[0], w_ref[...],
                         preferred_element_type=jnp.float32)


def _out_stage(out_part, L_out_s):
    bn = 1000
    return pl.pallas_call(
        _out_kernel,
        grid=(N // bn,),
        in_specs=[pl.BlockSpec((1, bn, D), lambda i: (i // 5, i % 5, 0)),
                  pl.BlockSpec((D, D), lambda i: (0, 0))],
        out_specs=pl.BlockSpec((bn, D), lambda i: (i, 0)),
        out_shape=jax.ShapeDtypeStruct((N, D), jnp.float32),
    )(out_part, L_out_s)


def kernel(edge_src, edge_dst, edge_scalar_attr, edge_weight_cutoff, edge_attr,
           node_feat, Wk1, Wk2, Pk, Wv1, Wv2, Pv, L_logit, L_out):
    W1k = Wk1 * (1.0 / jnp.sqrt(DS))
    W2k = Wk2 * (1.0 / jnp.sqrt(NW))
    W1v = Wv1 * (1.0 / jnp.sqrt(DS))
    W2v = Wv2 * (1.0 / jnp.sqrt(NW))
    Pk_t = jnp.transpose(Pk, (2, 0, 1))
    Pv_t = jnp.transpose(Pv, (2, 0, 1))
    Lt2 = jnp.transpose(L_logit, (1, 2, 0)).reshape(D, H * D)
    L_out_s = L_out * (1.0 / jnp.sqrt(D))

    srcf, dstf = _kg_stage(edge_src, edge_dst, node_feat)

    ev, expq = _edge_stage(edge_scalar_attr, edge_attr, srcf, dstf,
                           W1k, W2k, W1v, W2v, Pk_t, Pv_t, Lt2)

    expq_flat = expq.reshape(E * H)

    dst_bits = lax.bitcast_convert_type(edge_dst, jnp.float32)
    s_part = _ks_stage(expq_flat, dst_bits)
    z_part, r_flat = _kz_stage(expq_flat, dst_bits, edge_weight_cutoff, s_part)
    scale_flat = _ksc_stage(r_flat, edge_weight_cutoff, dst_bits, z_part)

    Bmat = jnp.kron(jnp.eye(H, dtype=jnp.float32),
                    jnp.ones((1, D // H), jnp.float32))
    evw = _mul_stage(ev, scale_flat.reshape(E, H), Bmat)
    idx_lo = jnp.where(edge_dst < NHALF, edge_dst, NHALF)
    idx_hi = jnp.where(edge_dst >= NHALF, edge_dst - NHALF, NHALF)
    idx2 = jnp.concatenate([idx_lo[None, :], idx_hi[None, :]], axis=0)
    out_part = _kw_stage(evw, idx2)

    return _out_stage(out_part, L_out_s)


# speedup vs baseline: 1.1180x; 1.0014x over previous
import functools

import jax
import jax.numpy as jnp
from jax import lax
from jax.experimental import pallas as pl
from jax.experimental.pallas import tpu as pltpu
from jax.experimental.pallas import tpu_sc as plsc

N = 10000
E = 160000
D = 128
DE = 4
DS = 16
NW = 64
H = 4

NHP = 40960
NWORK = 32
BE = 2000

CE1 = 5000
NC1 = E // CE1
CPW1 = NC1 // NWORK
CE2 = 128
NC2 = E // CE2


def _mesh():
    return plsc.VectorSubcoreMesh(core_axis_name="c", subcore_axis_name="s")


def _lanes():
    return lax.iota(jnp.int32, 16)


def _zero_1d(ref, n):
    z = jnp.zeros((16,), jnp.float32)

    def body(i):
        ref[pl.ds(i * 16, 16)] = z

    plsc.parallel_loop(0, n // 16, unroll=8)(body)


def _rsqrt_quake(x):
    i = plsc.bitcast(x, jnp.int32)
    i = 0x5F3759DF - lax.shift_right_arithmetic(i, 1)
    y = plsc.bitcast(i, jnp.float32)
    for _ in range(3):
        y = y * (1.5 - 0.5 * x * y * y)
    return y


RBS = 8192
RSUB = RBS // 16


def _block_reduce(acc, shared, tmp, red, out_at_c, s):
    for b in range(NHP // RBS):
        pltpu.sync_copy(acc.at[pl.ds(b * RBS, RBS)], shared.at[s])
        plsc.subcore_barrier()
        off = s * RSUB
        _zero_1d(red, RSUB)
        for j in range(16):
            pltpu.sync_copy(shared.at[j, pl.ds(off, RSUB)], tmp)

            def radd(i):
                red[pl.ds(i * 16, 16)] = (red[pl.ds(i * 16, 16)]
                                          + tmp[pl.ds(i * 16, 16)])

            plsc.parallel_loop(0, RSUB // 16, unroll=8)(radd)
        pltpu.sync_copy(red, out_at_c.at[pl.ds(b * RBS + off, RSUB)])
        plsc.subcore_barrier()


def _kg_body(srci_hbm, dsti_hbm, nf_hbm, srcf_out, dstf_out,
             idxa, buf0, buf1, sem0, sem1):
    c = lax.axis_index("c")
    s = lax.axis_index("s")
    wid = c * 16 + s
    t = lax.shift_right_arithmetic(wid, 4)
    g = lax.bitwise_and(wid, 15)
    EPW = E // 16
    NF = EPW // CE2
    base = g * EPW

    def run(idx_hbm, out_hbm):
        pltpu.sync_copy(idx_hbm.at[pl.ds(base, EPW)], idxa)

        def pair(t2, _):
            oa = (2 * t2) * CE2
            ob = (2 * t2 + 1) * CE2
            a = pltpu.async_copy(nf_hbm.at[idxa.at[pl.ds(oa, CE2)]], buf0, sem0)
            b = pltpu.async_copy(nf_hbm.at[idxa.at[pl.ds(ob, CE2)]], buf1, sem1)
            a.wait()
            pltpu.sync_copy(buf0, out_hbm.at[pl.ds(base + oa, CE2), :])
            b.wait()
            pltpu.sync_copy(buf1, out_hbm.at[pl.ds(base + ob, CE2), :])
            return 0

        lax.fori_loop(0, NF // 2, pair, 0)
        o = NF * CE2
        pltpu.async_copy(nf_hbm.at[idxa.at[pl.ds(o, EPW - NF * CE2)]],
                         buf0.at[pl.ds(0, EPW - NF * CE2), :], sem0).wait()
        pltpu.sync_copy(buf0.at[pl.ds(0, EPW - NF * CE2), :],
                        out_hbm.at[pl.ds(base + o, EPW - NF * CE2), :])

    @pl.when(t == 0)
    def _():
        run(srci_hbm, srcf_out)

    @pl.when(t == 1)
    def _():
        run(dsti_hbm, dstf_out)


def _kg_stage(edge_src, edge_dst, node_feat):
    f = functools.partial(
        pl.kernel,
        mesh=_mesh(),
        compiler_params=pltpu.CompilerParams(needs_layout_passes=False),
        out_type=[
            jax.ShapeDtypeStruct((E, D), jnp.float32),
            jax.ShapeDtypeStruct((E, D), jnp.float32),
        ],
        scratch_types=[
            pltpu.VMEM((E // 16,), jnp.int32),
            pltpu.VMEM((CE2, D), jnp.float32),
            pltpu.VMEM((CE2, D), jnp.float32),
            pltpu.SemaphoreType.DMA,
            pltpu.SemaphoreType.DMA,
        ],
    )
    return f(_kg_body)(edge_src, edge_dst, node_feat)


def _ks_body(expq_hbm, dst_hbm, s_out, acc, dstv, qv, tmp, red, shared):
    c = lax.axis_index("c")
    s = lax.axis_index("s")
    wid = c * 16 + s
    lanes = _lanes()
    _zero_1d(acc, NHP)

    def chunk(ci, _):
        base = (wid * CPW1 + ci) * CE1
        pltpu.sync_copy(dst_hbm.at[pl.ds(base, CE1)], dstv)
        pltpu.sync_copy(expq_hbm.at[pl.ds(base * H, CE1 * H)], qv)

        def grp(g):
            p = g * 16 + lanes
            el = lax.shift_right_arithmetic(p, 2)
            h = lax.bitwise_and(p, 3)
            d = plsc.bitcast(plsc.load_gather(dstv, [el]), jnp.int32)
            idx = d * 4 + h
            val = qv[pl.ds(g * 16, 16)]
            plsc.addupdate_scatter(acc, [idx], val)

        plsc.parallel_loop(0, CE1 * H // 16, unroll=4)(grp)
        return 0

    lax.fori_loop(0, CPW1, chunk, 0)

    _block_reduce(acc, shared, tmp, red, s_out.at[c], s)


def _ks_stage(expq_flat, dst_flat):
    f = functools.partial(
        pl.kernel,
        mesh=_mesh(),
        compiler_params=pltpu.CompilerParams(needs_layout_passes=False),
        out_type=jax.ShapeDtypeStruct((2, NHP), jnp.float32),
        scratch_types=[
            pltpu.VMEM((NHP,), jnp.float32),
            pltpu.VMEM((CE1,), jnp.float32),
            pltpu.VMEM((CE1 * H,), jnp.float32),
            pltpu.VMEM((RSUB,), jnp.float32),
            pltpu.VMEM((RSUB,), jnp.float32),
            pltpu.VMEM_SHARED((16, RBS), jnp.float32),
        ],
    )
    return f(_ks_body)(expq_flat, dst_flat)


def _kz_body(expq_hbm, dst_hbm, cut_hbm, s_part, z_out, r_out,
             stab, zacc, dstv, qv, cutv, tmp, red, shared):
    c = lax.axis_index("c")
    s = lax.axis_index("s")
    wid = c * 16 + s
    lanes = _lanes()

    pltpu.sync_copy(s_part.at[0], stab)
    for b in range(NHP // RSUB):
        pltpu.sync_copy(s_part.at[1, pl.ds(b * RSUB, RSUB)], tmp)

        def badd(i):
            o = b * RSUB + i * 16
            stab[pl.ds(o, 16)] = stab[pl.ds(o, 16)] + tmp[pl.ds(i * 16, 16)]

        plsc.parallel_loop(0, RSUB // 16, unroll=8)(badd)

    _zero_1d(zacc, NHP)

    def chunk(ci, _):
        base = (wid * CPW1 + ci) * CE1
        pltpu.sync_copy(dst_hbm.at[pl.ds(base, CE1)], dstv)
        pltpu.sync_copy(expq_hbm.at[pl.ds(base * H, CE1 * H)], qv)
        pltpu.sync_copy(cut_hbm.at[pl.ds(base, CE1)], cutv)

        def grp(g):
            p = g * 16 + lanes
            el = lax.shift_right_arithmetic(p, 2)
            h = lax.bitwise_and(p, 3)
            d = plsc.bitcast(plsc.load_gather(dstv, [el]), jnp.int32)
            idx = d * 4 + h
            sv = plsc.load_gather(stab, [idx])
            q = qv[pl.ds(g * 16, 16)]
            r = q / sv
            qv[pl.ds(g * 16, 16)] = r
            r2 = r * r
            cu = plsc.load_gather(cutv, [el])
            plsc.addupdate_scatter(zacc, [idx], cu * r2 * r2)

        plsc.parallel_loop(0, CE1 * H // 16, unroll=4)(grp)
        pltpu.sync_copy(qv, r_out.at[pl.ds(base * H, CE1 * H)])
        return 0

    lax.fori_loop(0, CPW1, chunk, 0)

    _block_reduce(zacc, shared, tmp, red, z_out.at[c], s)


def _kz_stage(expq_flat, dst_flat, cut, s_part):
    f = functools.partial(
        pl.kernel,
        mesh=_mesh(),
        compiler_params=pltpu.CompilerParams(needs_layout_passes=False),
        out_type=[
            jax.ShapeDtypeStruct((2, NHP), jnp.float32),
            jax.ShapeDtypeStruct((E * H,), jnp.float32),
        ],
        scratch_types=[
            pltpu.VMEM((NHP,), jnp.float32),
            pltpu.VMEM((NHP,), jnp.float32),
            pltpu.VMEM((CE1,), jnp.float32),
            pltpu.VMEM((CE1 * H,), jnp.float32),
            pltpu.VMEM((CE1,), jnp.float32),
            pltpu.VMEM((RSUB,), jnp.float32),
            pltpu.VMEM((RSUB,), jnp.float32),
            pltpu.VMEM_SHARED((16, RBS), jnp.float32),
        ],
    )
    return f(_kz_body)(expq_flat, dst_flat, cut, s_part)


def _ksc_body(r_hbm, cut_hbm, dst_hbm, z_part, scale_out,
              ztab, tmp, rv, cutv, dstv):
    c = lax.axis_index("c")
    s = lax.axis_index("s")
    wid = c * 16 + s
    lanes = _lanes()

    pltpu.sync_copy(z_part.at[0], ztab)
    ZB = 4096
    for b in range(NHP // ZB):
        pltpu.sync_copy(z_part.at[1, pl.ds(b * ZB, ZB)], tmp)

        def badd(i):
            o = b * ZB + i * 16
            x = ztab[pl.ds(o, 16)] + tmp[pl.ds(i * 16, 16)]
            xc = jnp.where(x == 0.0, 1.0, x)
            ztab[pl.ds(o, 16)] = _rsqrt_quake(xc)

        plsc.parallel_loop(0, ZB // 16, unroll=4)(badd)

    def chunk(ci, _):
        base = (wid * CPW1 + ci) * CE1
        pltpu.sync_copy(r_hbm.at[pl.ds(base * H, CE1 * H)], rv)
        pltpu.sync_copy(cut_hbm.at[pl.ds(base, CE1)], cutv)
        pltpu.sync_copy(dst_hbm.at[pl.ds(base, CE1)], dstv)

        def grp(g):
            p = g * 16 + lanes
            el = lax.shift_right_arithmetic(p, 2)
            h = lax.bitwise_and(p, 3)
            d = plsc.bitcast(plsc.load_gather(dstv, [el]), jnp.int32)
            iz = plsc.load_gather(ztab, [d * 4 + h])
            cu = plsc.load_gather(cutv, [el])
            sq = cu * _rsqrt_quake(jnp.where(cu == 0.0, 1.0, cu))
            rr = rv[pl.ds(g * 16, 16)]
            rv[pl.ds(g * 16, 16)] = rr * rr * sq * iz

        plsc.parallel_loop(0, CE1 * H // 16, unroll=4)(grp)
        pltpu.sync_copy(rv, scale_out.at[pl.ds(base * H, CE1 * H)])
        return 0

    lax.fori_loop(0, CPW1, chunk, 0)


def _ksc_stage(r_flat, cut, dst_bits, z_part):
    f = functools.partial(
        pl.kernel,
        mesh=_mesh(),
        compiler_params=pltpu.CompilerParams(needs_layout_passes=False),
        out_type=jax.ShapeDtypeStruct((E * H,), jnp.float32),
        scratch_types=[
            pltpu.VMEM((NHP,), jnp.float32),
            pltpu.VMEM((4096,), jnp.float32),
            pltpu.VMEM((CE1 * H,), jnp.float32),
            pltpu.VMEM((CE1,), jnp.float32),
            pltpu.VMEM((CE1,), jnp.float32),
        ],
    )
    return f(_ksc_body)(r_flat, cut, dst_bits, z_part)


NHALF = 5000
NROWS = 5120


def _kw_body(evw_hbm, idx2_hbm, out_part, evv, idxv, evv1, idxv1,
             sem0, sem1, nodeacc):
    c = lax.axis_index("c")
    s = lax.axis_index("s")
    zv = jnp.zeros((16,), jnp.float32)

    def zb(t):
        evv[lax.shift_right_arithmetic(t, 3),
            pl.ds(lax.bitwise_and(t, 7) * 16, 16)] = zv

    plsc.parallel_loop(0, 128 * 8, unroll=8)(zb)
    for k, nr in ((0, 128), (1, 128), (2, 64)):
        pltpu.sync_copy(evv.at[pl.ds(0, nr), :],
                        nodeacc.at[pl.ds(s * 320 + k * 128, nr), :])
    plsc.subcore_barrier()

    def pair(t, _):
        base_a = ((2 * t) * 16 + s) * CE2
        base_b = ((2 * t + 1) * 16 + s) * CE2
        a0 = pltpu.async_copy(evw_hbm.at[pl.ds(base_a, CE2), :], evv, sem0)
        a1 = pltpu.async_copy(idx2_hbm.at[c, pl.ds(base_a, CE2)], idxv, sem0)
        b0 = pltpu.async_copy(evw_hbm.at[pl.ds(base_b, CE2), :], evv1, sem1)
        b1 = pltpu.async_copy(idx2_hbm.at[c, pl.ds(base_b, CE2)], idxv1, sem1)
        a0.wait()
        a1.wait()
        pltpu.sync_copy(evv, nodeacc.at[idxv], add=True)
        b0.wait()
        b1.wait()
        pltpu.sync_copy(evv1, nodeacc.at[idxv1], add=True)
        return 0

    lax.fori_loop(0, NC2 // 32, pair, 0)

    @pl.when(s < NC2 % 16)
    def _():
        base = ((NC2 // 16) * 16 + s) * CE2
        pltpu.sync_copy(evw_hbm.at[pl.ds(base, CE2), :], evv)
        pltpu.sync_copy(idx2_hbm.at[c, pl.ds(base, CE2)], idxv)
        pltpu.sync_copy(evv, nodeacc.at[idxv], add=True)

    plsc.subcore_barrier()
    for k, nr in ((0, 128), (1, 128), (2, 64)):
        o = s * 320 + k * 128
        pltpu.sync_copy(nodeacc.at[pl.ds(o, nr), :],
                        out_part.at[c, pl.ds(o, nr), :])


def _kw_stage(evw, idx2):
    f = functools.partial(
        pl.kernel,
        mesh=_mesh(),
        compiler_params=pltpu.CompilerParams(needs_layout_passes=False),
        out_type=jax.ShapeDtypeStruct((2, NROWS, D), jnp.float32),
        scratch_types=[
            pltpu.VMEM((CE2, D), jnp.float32),
            pltpu.VMEM((CE2,), jnp.int32),
            pltpu.VMEM((CE2, D), jnp.float32),
            pltpu.VMEM((CE2,), jnp.int32),
            pltpu.SemaphoreType.DMA,
            pltpu.SemaphoreType.DMA,
            pltpu.VMEM_SHARED((NROWS, D), jnp.float32),
        ],
    )
    return f(_kw_body)(evw, idx2)


def _edge_kernel(esa_ref, eattr_ref, srcf_ref, dstf_ref,
                 W1k_ref, W2k_ref, W1v_ref, W2v_ref, Pk_ref, Pv_ref, Lt_ref,
                 ev_ref, expq_ref):
    esa = esa_ref[...]
    f32 = jnp.float32
    wk = jnp.maximum(esa @ W1k_ref[...], 0.0)
    wk = jnp.maximum(jnp.dot(wk, W2k_ref[...], preferred_element_type=f32), 0.0)
    wv = jnp.maximum(esa @ W1v_ref[...], 0.0)
    wv = jnp.maximum(jnp.dot(wv, W2v_ref[...], preferred_element_type=f32), 0.0)
    eattr = eattr_ref[...]
    srcf = srcf_ref[...]
    tk = None
    tv = None
    for v in range(DE):
        av = eattr[:, v:v + 1]
        pk = jnp.dot(wk, Pk_ref[v], preferred_element_type=f32)
        pv = jnp.dot(wv, Pv_ref[v], preferred_element_type=f32)
        tk = pk * av if tk is None else tk + pk * av
        tv = pv * av if tv is None else tv + pv * av
    ek = srcf * tk * (1.0 / 16.0)
    ev = srcf * tv * (1.0 / 16.0)
    ev_ref[...] = ev
    K2 = jnp.dot(ek, Lt_ref[...], preferred_element_type=f32)
    dstf = dstf_ref[...]
    cols = []
    for h in range(H):
        lg = jnp.sum(dstf * K2[:, h * D:(h + 1) * D], axis=1, keepdims=True)
        cols.append(jnp.exp(lg * (0.25 / D)))
    expq_ref[...] = jnp.concatenate(cols, axis=1)


def _edge_stage(esa, eattr, srcf, dstf, W1k, W2k, W1v, W2v, Pk_t, Pv_t, Lt2):
    nb = E // BE
    full = lambda shape: pl.BlockSpec(shape, lambda i: (0,) * len(shape))
    return pl.pallas_call(
        _edge_kernel,
        grid=(nb,),
        in_specs=[
            pl.BlockSpec((BE, DS), lambda i: (i, 0)),
            pl.BlockSpec((BE, DE), lambda i: (i, 0)),
            pl.BlockSpec((BE, D), lambda i: (i, 0)),
            pl.BlockSpec((BE, D), lambda i: (i, 0)),
            full((DS, NW)), full((NW, NW)), full((DS, NW)), full((NW, NW)),
            full((DE, NW, D)), full((DE, NW, D)), full((D, H * D)),
        ],
        out_specs=[
            pl.BlockSpec((BE, D), lambda i: (i, 0)),
            pl.BlockSpec((BE, H), lambda i: (i, 0)),
        ],
        out_shape=[
            jax.ShapeDtypeStruct((E, D), jnp.float32),
            jax.ShapeDtypeStruct((E, H), jnp.float32),
        ],
        compiler_params=pltpu.CompilerParams(
            dimension_semantics=("arbitrary",)),
    )(esa, eattr, srcf, dstf, W1k, W2k, W1v, W2v, Pk_t, Pv_t, Lt2)


def _mul_kernel(ev_ref, scale_ref, B_ref, evw_ref):
    sc = jnp.dot(scale_ref[...], B_ref[...],
                 preferred_element_type=jnp.float32)
    evw_ref[...] = ev_ref[...] * sc


def _mul_stage(ev, scale2, Bmat):
    nb = E // BE
    return pl.pallas_call(
        _mul_kernel,
        grid=(nb,),
        in_specs=[
            pl.BlockSpec((BE, D), lambda i: (i, 0)),
            pl.BlockSpec((BE, H), lambda i: (i, 0)),
            pl.BlockSpec((H, D), lambda i: (0, 0)),
        ],
        out_specs=pl.BlockSpec((BE, D), lambda i: (i, 0)),
        out_shape=jax.ShapeDtypeStruct((E, D), jnp.float32),
        compiler_params=pltpu.CompilerParams(
            dimension_semantics=("arbitrary",)),
    )(ev, scale2, Bmat)


def _out_kernel(p_ref, w_ref, o_ref):
    o_ref[...] = jnp.dot(p_ref[0], w_ref[...],
                         preferred_element_type=jnp.float32)


def _out_stage(out_part, L_out_s):
    bn = 1000
    return pl.pallas_call(
        _out_kernel,
        grid=(N // bn,),
        in_specs=[pl.BlockSpec((1, bn, D), lambda i: (i // 5, i % 5, 0)),
                  pl.BlockSpec((D, D), lambda i: (0, 0))],
        out_specs=pl.BlockSpec((bn, D), lambda i: (i, 0)),
        out_shape=jax.ShapeDtypeStruct((N, D), jnp.float32),
    )(out_part, L_out_s)


def kernel(edge_src, edge_dst, edge_scalar_attr, edge_weight_cutoff, edge_attr,
           node_feat, Wk1, Wk2, Pk, Wv1, Wv2, Pv, L_logit, L_out):
    W1k = Wk1 * (1.0 / jnp.sqrt(DS))
    W2k = Wk2 * (1.0 / jnp.sqrt(NW))
    W1v = Wv1 * (1.0 / jnp.sqrt(DS))
    W2v = Wv2 * (1.0 / jnp.sqrt(NW))
    Pk_t = jnp.transpose(Pk, (2, 0, 1))
    Pv_t = jnp.transpose(Pv, (2, 0, 1))
    Lt2 = jnp.transpose(L_logit, (1, 2, 0)).reshape(D, H * D)
    L_out_s = L_out * (1.0 / jnp.sqrt(D))

    srcf, dstf = _kg_stage(edge_src, edge_dst, node_feat)

    ev, expq = _edge_stage(edge_scalar_attr, edge_attr, srcf, dstf,
                           W1k, W2k, W1v, W2v, Pk_t, Pv_t, Lt2)

    expq_flat = expq.reshape(E * H)

    dst_bits = lax.bitcast_convert_type(edge_dst, jnp.float32)
    s_part = _ks_stage(expq_flat, dst_bits)
    z_part, r_flat = _kz_stage(expq_flat, dst_bits, edge_weight_cutoff, s_part)
    scale_flat = _ksc_stage(r_flat, edge_weight_cutoff, dst_bits, z_part)

    Bmat = jnp.kron(jnp.eye(H, dtype=jnp.float32),
                    jnp.ones((1, D // H), jnp.float32))
    evw = _mul_stage(ev, scale_flat.reshape(E, H), Bmat)
    idx_lo = jnp.where(edge_dst < NHALF, edge_dst, NHALF)
    idx_hi = jnp.where(edge_dst >= NHALF, edge_dst - NHALF, NHALF)
    idx2 = jnp.concatenate([idx_lo[None, :], idx_hi[None, :]], axis=0)
    out_part = _kw_stage(evw, idx2)

    return _out_stage(out_part, L_out_s)
